# baseline (speedup 1.0000x reference)
import functools
import math

import jax
import jax.numpy as jnp
from jax import lax
from jax.experimental import pallas as pl
from jax.experimental.pallas import tpu as pltpu
from jax.experimental.pallas import tpu_sc as plsc

NC = 2
NS = 16
NW = NC * NS
ECHUNK = 128


def _npad(n):
    return (n // 128 + 1) * 128


def _make_agg(n_nodes, half, chunks):
    npad = _npad(n_nodes)
    rows_per_tile = npad // NS
    mesh = plsc.VectorSubcoreMesh(core_axis_name="c", subcore_axis_name="s")

    @functools.partial(
        pl.kernel,
        out_type=jax.ShapeDtypeStruct((NC, npad, half), jnp.float32),
        mesh=mesh,
        scratch_types=[
            pltpu.VMEM((chunks, ECHUNK), jnp.int32),
            pltpu.VMEM((chunks, ECHUNK), jnp.int32),
            pltpu.VMEM((3, ECHUNK, half), jnp.float32),
            pltpu.VMEM_SHARED((npad, half), jnp.float32),
            pltpu.SemaphoreType.DMA,
            pltpu.SemaphoreType.DMA,
            pltpu.SemaphoreType.DMA,
        ],
        compiler_params=pltpu.CompilerParams(use_tc_tiling_on_sc=False),
    )
    def agg(h2_hbm, src_hbm, dst_hbm, zero_hbm, out_hbm,
            src_v, dst_v, rows_v, acc_sh, sem0, sem1, sem2):
        c = lax.axis_index("c")
        s = lax.axis_index("s")
        sems = (sem0, sem1, sem2)

        pltpu.sync_copy(src_hbm.at[c * NS + s], src_v)
        pltpu.sync_copy(dst_hbm.at[s], dst_v)
        for b in range(3):
            pltpu.async_copy(h2_hbm.at[src_v.at[b]], rows_v.at[b], sems[b])
        row0 = s * rows_per_tile
        pltpu.sync_copy(zero_hbm.at[pl.ds(row0, rows_per_tile)],
                        acc_sh.at[pl.ds(row0, rows_per_tile)])
        plsc.subcore_barrier()

        def triple(p, carry):
            for b in range(3):
                j = 3 * p + b
                pltpu.make_async_copy(h2_hbm.at[src_v.at[j]],
                                      rows_v.at[b], sems[b]).wait()
                pltpu.sync_copy(rows_v.at[b], acc_sh.at[dst_v.at[j]],
                                add=True)
                nxt = j + 3

                @pl.when(nxt < chunks)
                def _():
                    pltpu.async_copy(h2_hbm.at[src_v.at[nxt]],
                                     rows_v.at[b], sems[b])
            return carry

        lax.fori_loop(0, chunks // 3, triple, 0)
        plsc.subcore_barrier()
        pltpu.sync_copy(acc_sh.at[pl.ds(row0, rows_per_tile)],
                        out_hbm.at[c, pl.ds(row0, rows_per_tile)])

    return agg


def _make_deg(n_nodes, chunks32):
    npad = _npad(n_nodes)
    colw = -(-(npad // NS) // 16) * 16
    npad2 = colw * NS
    mesh = plsc.VectorSubcoreMesh(core_axis_name="c", subcore_axis_name="s")

    @functools.partial(
        pl.kernel,
        out_type=jax.ShapeDtypeStruct((NC, npad2), jnp.float32),
        mesh=mesh,
        scratch_types=[
            pltpu.VMEM((chunks32, ECHUNK), jnp.int32),
            pltpu.VMEM((npad2,), jnp.float32),
            pltpu.VMEM((NS, colw), jnp.float32),
            pltpu.VMEM_SHARED((NS, npad2), jnp.float32),
        ],
        compiler_params=pltpu.CompilerParams(use_tc_tiling_on_sc=False,
                                             needs_layout_passes=False),
    )
    def deg(dst_hbm, out_hbm, dst_v, part_v, red_v, shared_sh):
        c = lax.axis_index("c")
        s = lax.axis_index("s")
        w = c * NS + s
        pltpu.sync_copy(dst_hbm.at[w], dst_v)

        zeros16 = jnp.zeros((16,), jnp.float32)
        ones16 = jnp.ones((16,), jnp.float32)

        def zbody(i, carry):
            part_v[pl.ds(i * 16, 16)] = zeros16
            return carry

        lax.fori_loop(0, npad2 // 16, zbody, 0)

        def cbody(j, carry):
            for k in range(ECHUNK // 16):
                idx = dst_v[j, pl.ds(k * 16, 16)]
                plsc.addupdate_scatter(part_v, [idx], ones16)
            return carry

        lax.fori_loop(0, chunks32, cbody, 0)
        pltpu.sync_copy(part_v, shared_sh.at[s])
        plsc.subcore_barrier()

        col0 = s * colw
        pltpu.sync_copy(shared_sh.at[:, pl.ds(col0, colw)], red_v)

        def rbody(k, carry):
            acc = red_v[0, pl.ds(k * 16, 16)]
            for rr in range(1, NS):
                acc = acc + red_v[rr, pl.ds(k * 16, 16)]
            part_v[pl.ds(k * 16, 16)] = acc
            return carry

        lax.fori_loop(0, colw // 16, rbody, 0)
        pltpu.sync_copy(part_v.at[pl.ds(0, colw)],
                        out_hbm.at[c, pl.ds(col0, colw)])

    return deg


def _silu(v):
    return v * (1.0 / (1.0 + jnp.exp(-v)))


def _rblk(n):
    for b in (2000, 1000, 500, 250, 200, 125, 100, 50, 40, 25, 20, 10, 8):
        if n % b == 0:
            return b
    return n


def _make_dense_stages(n, d):
    r = _rblk(n)
    grid = (n // r,)
    f32 = jnp.float32
    half = d // 2

    def rows(h):
        return pl.BlockSpec((r, h), lambda i: (i, 0))

    def urows(h):
        return pl.BlockSpec((2, r, h), lambda i: (0, i, 0))


    def full(*shape):
        return pl.BlockSpec(shape, lambda i: tuple(0 for _ in shape))

    def dis_blk(deg_ref):
        return lax.rsqrt(deg_ref[...] + 1.0)

    def dot(a, b):
        return jnp.dot(a, b, preferred_element_type=f32)

    def s0_body(t_ref, noise_ref, wt0, bt0, wt1, bt1, wd0, q1_ref):
        tt = t_ref[...].astype(f32)
        k = lax.broadcasted_iota(jnp.int32, (1, half), 1).astype(f32)
        freq = jnp.exp(k * (-math.log(10000.0) / (half - 1)))
        ang = tt * freq
        emb = jnp.concatenate([jnp.sin(ang), jnp.cos(ang)], axis=1)
        e1 = _silu(dot(emb, wt0[...]) + bt0[...])
        x_t = noise_ref[...] + dot(e1, wt1[...]) + bt1[...]
        q1_ref[...] = dot(x_t, wd0[...])

    s0 = pl.pallas_call(
        s0_body,
        grid=grid,
        in_specs=[rows(1), rows(d), full(d, d), full(1, d),
                  full(d, d), full(1, d), full(d, d)],
        out_specs=rows(d),
        out_shape=jax.ShapeDtypeStruct((n, d), f32),
    )

    def s1_body(u_ref, p1_ref, degp_ref, bd0, wd1, h1_ref, p2_ref):
        dis = dis_blk(degp_ref)
        ucat = jnp.concatenate([u_ref[0], u_ref[1]], axis=1)
        h1 = _silu(dis * (ucat + p1_ref[...]) + bd0[...])
        h1_ref[...] = h1
        p2_ref[...] = dis * dot(h1, wd1[...])

    s1 = pl.pallas_call(
        s1_body,
        grid=grid,
        in_specs=[urows(half), rows(d), rows(1), full(1, d), full(d, half)],
        out_specs=[rows(d), rows(half)],
        out_shape=[jax.ShapeDtypeStruct((n, d), f32),
                   jax.ShapeDtypeStruct((n, half), f32)],
    )

    def s2_body(u_ref, p2_ref, degp_ref, bd1, p3_ref):
        dis = dis_blk(degp_ref)
        ucat = jnp.concatenate([u_ref[0], u_ref[1]], axis=1)
        h2 = _silu(dis * (ucat + p2_ref[...]) + bd1[...])
        p3_ref[...] = dis * h2

    s2 = pl.pallas_call(
        s2_body,
        grid=grid,
        in_specs=[urows(half // 2), rows(half), rows(1), full(1, half)],
        out_specs=rows(half),
        out_shape=jax.ShapeDtypeStruct((n, half), f32),
    )

    def s3_body(u_ref, p3_ref, h1_ref, degp_ref, wu0, bu0, wu1a, wu1b,
                p4_ref):
        dis = dis_blk(degp_ref)
        ucat = jnp.concatenate([u_ref[0], u_ref[1]], axis=1)
        h3 = _silu(dis * dot(ucat + p3_ref[...], wu0[...]) + bu0[...])
        p4_ref[...] = dis * (dot(h3, wu1a[...]) + dot(h1_ref[...], wu1b[...]))

    s3 = pl.pallas_call(
        s3_body,
        grid=grid,
        in_specs=[urows(half // 2), rows(half), rows(d), rows(1),
                  full(half, d), full(1, d), full(d, d), full(d, d)],
        out_specs=rows(d),
        out_shape=jax.ShapeDtypeStruct((n, d), f32),
    )

    def s4_body(u_ref, p4_ref, degp_ref, bu1, out_ref):
        dis = dis_blk(degp_ref)
        ucat = jnp.concatenate([u_ref[0], u_ref[1]], axis=1)
        out_ref[...] = _silu(dis * (ucat + p4_ref[...]) + bu1[...])

    s4 = pl.pallas_call(
        s4_body,
        grid=grid,
        in_specs=[urows(half), rows(d), rows(1), full(1, d)],
        out_specs=rows(d),
        out_shape=jax.ShapeDtypeStruct((n, d), f32),
    )

    return s0, s1, s2, s3, s4


def kernel(x, noise_graph_X_t, edge_index, t,
           W_t0, b_t0, W_t1, b_t1,
           W_d0, b_d0, W_d1, b_d1,
           W_u0, b_u0, W_u1, b_u1):
    n = x.shape[0]
    d = x.shape[1]
    e = edge_index.shape[1]
    npad = _npad(n)

    chunks = 3 * (-(-e // (NS * ECHUNK * 3)))
    ep = NS * ECHUNK * chunks
    pad = ep - e
    src = jnp.concatenate([edge_index[0], jnp.zeros((pad,), jnp.int32)])
    dst = jnp.concatenate([edge_index[1], jnp.full((pad,), n, jnp.int32)])
    src2 = (2 * src)[None, :] + jnp.arange(2, dtype=jnp.int32)[:, None]
    src2 = src2.reshape(2 * NS, chunks, ECHUNK)
    dst16 = dst.reshape(NS, chunks, ECHUNK)

    chunks32 = -(-e // (NW * ECHUNK))
    epd = NW * ECHUNK * chunks32
    dst32 = jnp.concatenate(
        [edge_index[1], jnp.full((epd - e,), n, jnp.int32)]
    ).reshape(NW, chunks32, ECHUNK)

    zeros64 = jnp.zeros((npad, d // 2), jnp.float32)
    zeros32 = jnp.zeros((npad, d // 4), jnp.float32)

    agg128 = _make_agg(n, d // 2, chunks)
    agg64 = _make_agg(n, d // 4, chunks)
    degk = _make_deg(n, chunks32)

    def agg(p, aggk, zeros):
        f = p.shape[1]
        return aggk(p.reshape(2 * n, f // 2), src2, dst16, zeros)

    s0, s1, s2, s3, s4 = _make_dense_stages(n, d)

    degout = degk(dst32)
    degn = (degout[0, :n] + degout[1, :n]).reshape(n, 1)

    row = lambda v: v.reshape(1, -1)
    q1 = s0(t.reshape(n, 1), noise_graph_X_t,
            W_t0, row(b_t0), W_t1, row(b_t1), W_d0)
    p1 = lax.rsqrt(degn + 1.0) * q1
    u1 = agg(p1, agg128, zeros64)
    h1, p2 = s1(u1, p1, degn, row(b_d0), W_d1)
    u2 = agg(p2, agg64, zeros32)
    p3 = s2(u2, p2, degn, row(b_d1))
    u3 = agg(p3, agg64, zeros32)
    p4 = s3(u3, p3, h1, degn, W_u0, row(b_u0), W_u1[:d], W_u1[d:])
    u4 = agg(p4, agg128, zeros64)
    return s4(u4, p4, degn, row(b_u1))


# speedup vs baseline: 1.1512x; 1.1512x over previous
import functools
import math

import jax
import jax.numpy as jnp
from jax import lax
from jax.experimental import pallas as pl
from jax.experimental.pallas import tpu as pltpu
from jax.experimental.pallas import tpu_sc as plsc

NC = 2
NS = 16
NW = NC * NS
ECHUNK = 128


def _npad(n):
    return (n // 128 + 1) * 128


def _make_agg(n_nodes, half, chunks):
    npad = _npad(n_nodes)
    rows_per_tile = npad // NS
    mesh = plsc.VectorSubcoreMesh(core_axis_name="c", subcore_axis_name="s")

    @functools.partial(
        pl.kernel,
        out_type=jax.ShapeDtypeStruct((NC, npad, half), jnp.float32),
        mesh=mesh,
        scratch_types=[
            pltpu.VMEM((chunks, ECHUNK), jnp.int32),
            pltpu.VMEM((chunks, ECHUNK), jnp.int32),
            pltpu.VMEM((2, ECHUNK, half), jnp.float32),
            pltpu.VMEM_SHARED((npad, half), jnp.float32),
            pltpu.SemaphoreType.DMA,
            pltpu.SemaphoreType.DMA,
        ],
        compiler_params=pltpu.CompilerParams(use_tc_tiling_on_sc=False),
    )
    def agg(h2_hbm, src_hbm, dst_hbm, zero_hbm, out_hbm,
            src_v, dst_v, rows_v, acc_sh, sem0, sem1):
        c = lax.axis_index("c")
        s = lax.axis_index("s")
        sems = (sem0, sem1)

        pltpu.sync_copy(src_hbm.at[c * NS + s], src_v)
        pltpu.sync_copy(dst_hbm.at[s], dst_v)
        for b in range(2):
            pltpu.async_copy(h2_hbm.at[src_v.at[b]], rows_v.at[b], sems[b])
        row0 = s * rows_per_tile
        pltpu.sync_copy(zero_hbm.at[pl.ds(row0, rows_per_tile)],
                        acc_sh.at[pl.ds(row0, rows_per_tile)])
        plsc.subcore_barrier()

        def pair(p, carry):
            for b in range(2):
                j = 2 * p + b
                pltpu.make_async_copy(h2_hbm.at[src_v.at[j]],
                                      rows_v.at[b], sems[b]).wait()
                pltpu.sync_copy(rows_v.at[b], acc_sh.at[dst_v.at[j]],
                                add=True)
                nxt = j + 2

                @pl.when(nxt < chunks)
                def _():
                    pltpu.async_copy(h2_hbm.at[src_v.at[nxt]],
                                     rows_v.at[b], sems[b])
            return carry

        lax.fori_loop(0, chunks // 2, pair, 0)
        plsc.subcore_barrier()
        pltpu.sync_copy(acc_sh.at[pl.ds(row0, rows_per_tile)],
                        out_hbm.at[c, pl.ds(row0, rows_per_tile)])

    return agg


def _make_deg(n_nodes, chunks32):
    npad = _npad(n_nodes)
    colw = -(-(npad // NS) // 16) * 16
    npad2 = colw * NS
    mesh = plsc.VectorSubcoreMesh(core_axis_name="c", subcore_axis_name="s")

    @functools.partial(
        pl.kernel,
        out_type=jax.ShapeDtypeStruct((NC, npad2), jnp.float32),
        mesh=mesh,
        scratch_types=[
            pltpu.VMEM((chunks32, ECHUNK), jnp.int32),
            pltpu.VMEM((npad2,), jnp.float32),
            pltpu.VMEM((NS, colw), jnp.float32),
            pltpu.VMEM_SHARED((NS, npad2), jnp.float32),
        ],
        compiler_params=pltpu.CompilerParams(use_tc_tiling_on_sc=False,
                                             needs_layout_passes=False),
    )
    def deg(dst_hbm, out_hbm, dst_v, part_v, red_v, shared_sh):
        c = lax.axis_index("c")
        s = lax.axis_index("s")
        w = c * NS + s
        pltpu.sync_copy(dst_hbm.at[w], dst_v)

        zeros16 = jnp.zeros((16,), jnp.float32)
        ones16 = jnp.ones((16,), jnp.float32)

        def zbody(i, carry):
            part_v[pl.ds(i * 16, 16)] = zeros16
            return carry

        lax.fori_loop(0, npad2 // 16, zbody, 0)

        def cbody(j, carry):
            for k in range(ECHUNK // 16):
                idx = dst_v[j, pl.ds(k * 16, 16)]
                plsc.addupdate_scatter(part_v, [idx], ones16)
            return carry

        lax.fori_loop(0, chunks32, cbody, 0)
        pltpu.sync_copy(part_v, shared_sh.at[s])
        plsc.subcore_barrier()

        col0 = s * colw
        pltpu.sync_copy(shared_sh.at[:, pl.ds(col0, colw)], red_v)

        def rbody(k, carry):
            acc = red_v[0, pl.ds(k * 16, 16)]
            for rr in range(1, NS):
                acc = acc + red_v[rr, pl.ds(k * 16, 16)]
            part_v[pl.ds(k * 16, 16)] = acc
            return carry

        lax.fori_loop(0, colw // 16, rbody, 0)
        pltpu.sync_copy(part_v.at[pl.ds(0, colw)],
                        out_hbm.at[c, pl.ds(col0, colw)])

    return deg


def _silu(v):
    return v * (1.0 / (1.0 + jnp.exp(-v)))


def _rblk(n):
    for b in (2000, 1000, 500, 250, 200, 125, 100, 50, 40, 25, 20, 10, 8):
        if n % b == 0:
            return b
    return n


def _make_dense_stages(n, d):
    r = _rblk(n)
    grid = (n // r,)
    f32 = jnp.float32
    half = d // 2

    def rows(h):
        return pl.BlockSpec((r, h), lambda i: (i, 0))

    def urows(h):
        return pl.BlockSpec((2, r, h), lambda i: (0, i, 0))


    def full(*shape):
        return pl.BlockSpec(shape, lambda i: tuple(0 for _ in shape))

    def dis_blk(deg_ref):
        return lax.rsqrt(deg_ref[...] + 1.0)

    def dot(a, b):
        return jnp.dot(a, b, preferred_element_type=f32)

    def s0_body(t_ref, noise_ref, wt0, bt0, wt1, bt1, wd0, q1_ref):
        tt = t_ref[...].astype(f32)
        k = lax.broadcasted_iota(jnp.int32, (1, half), 1).astype(f32)
        freq = jnp.exp(k * (-math.log(10000.0) / (half - 1)))
        ang = tt * freq
        emb = jnp.concatenate([jnp.sin(ang), jnp.cos(ang)], axis=1)
        e1 = _silu(dot(emb, wt0[...]) + bt0[...])
        x_t = noise_ref[...] + dot(e1, wt1[...]) + bt1[...]
        q1_ref[...] = dot(x_t, wd0[...])

    s0 = pl.pallas_call(
        s0_body,
        grid=grid,
        in_specs=[rows(1), rows(d), full(d, d), full(1, d),
                  full(d, d), full(1, d), full(d, d)],
        out_specs=rows(d),
        out_shape=jax.ShapeDtypeStruct((n, d), f32),
    )

    def s1_body(u_ref, p1_ref, degp_ref, bd0, wd1, h1_ref, p2_ref):
        dis = dis_blk(degp_ref)
        ucat = jnp.concatenate([u_ref[0], u_ref[1]], axis=1)
        h1 = _silu(dis * (ucat + p1_ref[...]) + bd0[...])
        h1_ref[...] = h1
        p2_ref[...] = dis * dot(h1, wd1[...])

    s1 = pl.pallas_call(
        s1_body,
        grid=grid,
        in_specs=[urows(half), rows(d), rows(1), full(1, d), full(d, half)],
        out_specs=[rows(d), rows(half)],
        out_shape=[jax.ShapeDtypeStruct((n, d), f32),
                   jax.ShapeDtypeStruct((n, half), f32)],
    )

    def s2_body(u_ref, p2_ref, degp_ref, bd1, p3_ref):
        dis = dis_blk(degp_ref)
        ucat = jnp.concatenate([u_ref[0], u_ref[1]], axis=1)
        h2 = _silu(dis * (ucat + p2_ref[...]) + bd1[...])
        p3_ref[...] = dis * h2

    s2 = pl.pallas_call(
        s2_body,
        grid=grid,
        in_specs=[urows(half // 2), rows(half), rows(1), full(1, half)],
        out_specs=rows(half),
        out_shape=jax.ShapeDtypeStruct((n, half), f32),
    )

    def s3_body(u_ref, p3_ref, h1_ref, degp_ref, wu0, bu0, wu1a, wu1b,
                p4_ref):
        dis = dis_blk(degp_ref)
        ucat = jnp.concatenate([u_ref[0], u_ref[1]], axis=1)
        h3 = _silu(dis * dot(ucat + p3_ref[...], wu0[...]) + bu0[...])
        p4_ref[...] = dis * (dot(h3, wu1a[...]) + dot(h1_ref[...], wu1b[...]))

    s3 = pl.pallas_call(
        s3_body,
        grid=grid,
        in_specs=[urows(half // 2), rows(half), rows(d), rows(1),
                  full(half, d), full(1, d), full(d, d), full(d, d)],
        out_specs=rows(d),
        out_shape=jax.ShapeDtypeStruct((n, d), f32),
    )

    def s4_body(u_ref, p4_ref, degp_ref, bu1, out_ref):
        dis = dis_blk(degp_ref)
        ucat = jnp.concatenate([u_ref[0], u_ref[1]], axis=1)
        out_ref[...] = _silu(dis * (ucat + p4_ref[...]) + bu1[...])

    s4 = pl.pallas_call(
        s4_body,
        grid=grid,
        in_specs=[urows(half), rows(d), rows(1), full(1, d)],
        out_specs=rows(d),
        out_shape=jax.ShapeDtypeStruct((n, d), f32),
    )

    return s0, s1, s2, s3, s4


def kernel(x, noise_graph_X_t, edge_index, t,
           W_t0, b_t0, W_t1, b_t1,
           W_d0, b_d0, W_d1, b_d1,
           W_u0, b_u0, W_u1, b_u1):
    n = x.shape[0]
    d = x.shape[1]
    e = edge_index.shape[1]
    npad = _npad(n)

    chunks = 2 * (-(-e // (NS * ECHUNK * 2)))
    ep = NS * ECHUNK * chunks
    pad = ep - e
    src = jnp.concatenate([edge_index[0], jnp.zeros((pad,), jnp.int32)])
    dst = jnp.concatenate([edge_index[1], jnp.full((pad,), n, jnp.int32)])
    src2 = (2 * src)[None, :] + jnp.arange(2, dtype=jnp.int32)[:, None]
    src2 = src2.reshape(2 * NS, chunks, ECHUNK)
    dst16 = dst.reshape(NS, chunks, ECHUNK)

    chunks32 = -(-e // (NW * ECHUNK))
    epd = NW * ECHUNK * chunks32
    dst32 = jnp.concatenate(
        [edge_index[1], jnp.full((epd - e,), n, jnp.int32)]
    ).reshape(NW, chunks32, ECHUNK)

    zeros64 = jnp.zeros((npad, d // 2), jnp.float32)
    zeros32 = jnp.zeros((npad, d // 4), jnp.float32)

    agg128 = _make_agg(n, d // 2, chunks)
    agg64 = _make_agg(n, d // 4, chunks)
    degk = _make_deg(n, chunks32)

    def agg(p, aggk, zeros):
        f = p.shape[1]
        return aggk(p.reshape(2 * n, f // 2), src2, dst16, zeros)

    s0, s1, s2, s3, s4 = _make_dense_stages(n, d)

    degout = degk(dst32)
    degn = (degout[0, :n] + degout[1, :n]).reshape(n, 1)

    row = lambda v: v.reshape(1, -1)
    q1 = s0(t.reshape(n, 1), noise_graph_X_t,
            W_t0, row(b_t0), W_t1, row(b_t1), W_d0)
    p1 = lax.rsqrt(degn + 1.0) * q1
    u1 = agg(p1, agg128, zeros64)
    h1, p2 = s1(u1, p1, degn, row(b_d0), W_d1)
    u2 = agg(p2, agg64, zeros32)
    p3 = s2(u2, p2, degn, row(b_d1))
    u3 = agg(p3, agg64, zeros32)
    p4 = s3(u3, p3, h1, degn, W_u0, row(b_u0), W_u1[:d], W_u1[d:])
    u4 = agg(p4, agg128, zeros64)
    return s4(u4, p4, degn, row(b_u1))


# speedup vs baseline: 1.2608x; 1.0952x over previous
import functools
import math

import jax
import jax.numpy as jnp
from jax import lax
from jax.experimental import pallas as pl
from jax.experimental.pallas import tpu as pltpu
from jax.experimental.pallas import tpu_sc as plsc

NC = 2
NS = 16
NW = NC * NS
ECHUNK = 128


def _npad(n):
    return (n // 128 + 1) * 128


def _make_agg(n_nodes, half, chunks):
    npad = _npad(n_nodes)
    rows_per_tile = npad // NS
    mesh = plsc.VectorSubcoreMesh(core_axis_name="c", subcore_axis_name="s")

    @functools.partial(
        pl.kernel,
        out_type=jax.ShapeDtypeStruct((NC, npad, half), jnp.float32),
        mesh=mesh,
        scratch_types=[
            pltpu.VMEM((chunks, ECHUNK), jnp.int32),
            pltpu.VMEM((chunks, ECHUNK), jnp.int32),
            pltpu.VMEM((2, ECHUNK, half), jnp.float32),
            pltpu.VMEM_SHARED((npad, half), jnp.float32),
            pltpu.SemaphoreType.DMA,
            pltpu.SemaphoreType.DMA,
        ],
        compiler_params=pltpu.CompilerParams(use_tc_tiling_on_sc=False),
    )
    def agg(h2_hbm, src_hbm, dst_hbm, zero_hbm, out_hbm,
            src_v, dst_v, rows_v, acc_sh, sem0, sem1):
        c = lax.axis_index("c")
        s = lax.axis_index("s")
        sems = (sem0, sem1)

        pltpu.sync_copy(src_hbm.at[c * NS + s], src_v)
        pltpu.sync_copy(dst_hbm.at[s], dst_v)
        for b in range(2):
            pltpu.async_copy(h2_hbm.at[src_v.at[b]], rows_v.at[b], sems[b])
        row0 = s * rows_per_tile
        pltpu.sync_copy(zero_hbm.at[pl.ds(row0, rows_per_tile)],
                        acc_sh.at[pl.ds(row0, rows_per_tile)])
        plsc.subcore_barrier()

        def pair(p, carry):
            for b in range(2):
                j = 2 * p + b
                pltpu.make_async_copy(h2_hbm.at[src_v.at[j]],
                                      rows_v.at[b], sems[b]).wait()
                pltpu.sync_copy(rows_v.at[b], acc_sh.at[dst_v.at[j]],
                                add=True)
                nxt = j + 2

                @pl.when(nxt < chunks)
                def _():
                    pltpu.async_copy(h2_hbm.at[src_v.at[nxt]],
                                     rows_v.at[b], sems[b])
            return carry

        lax.fori_loop(0, chunks // 2, pair, 0)
        plsc.subcore_barrier()
        pltpu.sync_copy(acc_sh.at[pl.ds(row0, rows_per_tile)],
                        out_hbm.at[c, pl.ds(row0, rows_per_tile)])

    return agg


def _make_agg_sp(n_nodes, half, chunks, tpad):
    npad = _npad(n_nodes)
    rows_per_tile = npad // NS
    tab_per_tile = tpad // NS
    mesh = plsc.VectorSubcoreMesh(core_axis_name="c", subcore_axis_name="s")

    @functools.partial(
        pl.kernel,
        out_type=jax.ShapeDtypeStruct((NC, npad, half), jnp.float32),
        mesh=mesh,
        scratch_types=[
            pltpu.VMEM((chunks, ECHUNK), jnp.int32),
            pltpu.VMEM((chunks, ECHUNK), jnp.int32),
            pltpu.VMEM((2, ECHUNK, half), jnp.float32),
            pltpu.VMEM_SHARED((tpad, half), jnp.float32),
            pltpu.VMEM_SHARED((npad, half), jnp.float32),
            pltpu.SemaphoreType.DMA,
            pltpu.SemaphoreType.DMA,
        ],
        compiler_params=pltpu.CompilerParams(use_tc_tiling_on_sc=False),
    )
    def agg(h2_hbm, src_hbm, dst_hbm, zero_hbm, out_hbm,
            src_v, dst_v, rows_v, tab_sh, acc_sh, sem0, sem1):
        c = lax.axis_index("c")
        s = lax.axis_index("s")
        sems = (sem0, sem1)

        pltpu.sync_copy(src_hbm.at[c * NS + s], src_v)
        pltpu.sync_copy(dst_hbm.at[s], dst_v)
        trow0 = s * tab_per_tile
        pltpu.sync_copy(h2_hbm.at[pl.ds(trow0, tab_per_tile)],
                        tab_sh.at[pl.ds(trow0, tab_per_tile)])
        row0 = s * rows_per_tile
        pltpu.sync_copy(zero_hbm.at[pl.ds(row0, rows_per_tile)],
                        acc_sh.at[pl.ds(row0, rows_per_tile)])
        plsc.subcore_barrier()
        for b in range(2):
            pltpu.async_copy(tab_sh.at[src_v.at[b]], rows_v.at[b], sems[b])

        def pair(p, carry):
            for b in range(2):
                j = 2 * p + b
                pltpu.make_async_copy(tab_sh.at[src_v.at[j]],
                                      rows_v.at[b], sems[b]).wait()
                pltpu.sync_copy(rows_v.at[b], acc_sh.at[dst_v.at[j]],
                                add=True)
                nxt = j + 2

                @pl.when(nxt < chunks)
                def _():
                    pltpu.async_copy(tab_sh.at[src_v.at[nxt]],
                                     rows_v.at[b], sems[b])
            return carry

        lax.fori_loop(0, chunks // 2, pair, 0)
        plsc.subcore_barrier()
        pltpu.sync_copy(acc_sh.at[pl.ds(row0, rows_per_tile)],
                        out_hbm.at[c, pl.ds(row0, rows_per_tile)])

    return agg


def _make_deg(n_nodes, chunks32):
    npad = _npad(n_nodes)
    colw = -(-(npad // NS) // 16) * 16
    npad2 = colw * NS
    mesh = plsc.VectorSubcoreMesh(core_axis_name="c", subcore_axis_name="s")

    @functools.partial(
        pl.kernel,
        out_type=jax.ShapeDtypeStruct((NC, npad2), jnp.float32),
        mesh=mesh,
        scratch_types=[
            pltpu.VMEM((chunks32, ECHUNK), jnp.int32),
            pltpu.VMEM((npad2,), jnp.float32),
            pltpu.VMEM((NS, colw), jnp.float32),
            pltpu.VMEM_SHARED((NS, npad2), jnp.float32),
        ],
        compiler_params=pltpu.CompilerParams(use_tc_tiling_on_sc=False,
                                             needs_layout_passes=False),
    )
    def deg(dst_hbm, out_hbm, dst_v, part_v, red_v, shared_sh):
        c = lax.axis_index("c")
        s = lax.axis_index("s")
        w = c * NS + s
        pltpu.sync_copy(dst_hbm.at[w], dst_v)

        zeros16 = jnp.zeros((16,), jnp.float32)
        ones16 = jnp.ones((16,), jnp.float32)

        def zbody(i, carry):
            part_v[pl.ds(i * 16, 16)] = zeros16
            return carry

        lax.fori_loop(0, npad2 // 16, zbody, 0)

        def cbody(j, carry):
            for k in range(ECHUNK // 16):
                idx = dst_v[j, pl.ds(k * 16, 16)]
                plsc.addupdate_scatter(part_v, [idx], ones16)
            return carry

        lax.fori_loop(0, chunks32, cbody, 0)
        pltpu.sync_copy(part_v, shared_sh.at[s])
        plsc.subcore_barrier()

        col0 = s * colw
        pltpu.sync_copy(shared_sh.at[:, pl.ds(col0, colw)], red_v)

        def rbody(k, carry):
            acc = red_v[0, pl.ds(k * 16, 16)]
            for rr in range(1, NS):
                acc = acc + red_v[rr, pl.ds(k * 16, 16)]
            part_v[pl.ds(k * 16, 16)] = acc
            return carry

        lax.fori_loop(0, colw // 16, rbody, 0)
        pltpu.sync_copy(part_v.at[pl.ds(0, colw)],
                        out_hbm.at[c, pl.ds(col0, colw)])

    return deg


def _silu(v):
    return v * (1.0 / (1.0 + jnp.exp(-v)))


def _rblk(n):
    for b in (2000, 1000, 500, 250, 200, 125, 100, 50, 40, 25, 20, 10, 8):
        if n % b == 0:
            return b
    return n


def _make_dense_stages(n, d):
    r = _rblk(n)
    grid = (n // r,)
    f32 = jnp.float32
    half = d // 2

    def rows(h):
        return pl.BlockSpec((r, h), lambda i: (i, 0))

    def urows(h):
        return pl.BlockSpec((2, r, h), lambda i: (0, i, 0))


    def full(*shape):
        return pl.BlockSpec(shape, lambda i: tuple(0 for _ in shape))

    def dis_blk(deg_ref):
        return lax.rsqrt(deg_ref[...] + 1.0)

    def dot(a, b):
        return jnp.dot(a, b, preferred_element_type=f32)

    def s0_body(t_ref, noise_ref, wt0, bt0, wt1, bt1, wd0, q1_ref):
        tt = t_ref[...].astype(f32)
        k = lax.broadcasted_iota(jnp.int32, (1, half), 1).astype(f32)
        freq = jnp.exp(k * (-math.log(10000.0) / (half - 1)))
        ang = tt * freq
        emb = jnp.concatenate([jnp.sin(ang), jnp.cos(ang)], axis=1)
        e1 = _silu(dot(emb, wt0[...]) + bt0[...])
        x_t = noise_ref[...] + dot(e1, wt1[...]) + bt1[...]
        q1_ref[...] = dot(x_t, wd0[...])

    s0 = pl.pallas_call(
        s0_body,
        grid=grid,
        in_specs=[rows(1), rows(d), full(d, d), full(1, d),
                  full(d, d), full(1, d), full(d, d)],
        out_specs=rows(d),
        out_shape=jax.ShapeDtypeStruct((n, d), f32),
    )

    def s1_body(u_ref, p1_ref, degp_ref, bd0, wd1, h1_ref, p2_ref):
        dis = dis_blk(degp_ref)
        ucat = jnp.concatenate([u_ref[0], u_ref[1]], axis=1)
        h1 = _silu(dis * (ucat + p1_ref[...]) + bd0[...])
        h1_ref[...] = h1
        p2_ref[...] = dis * dot(h1, wd1[...])

    s1 = pl.pallas_call(
        s1_body,
        grid=grid,
        in_specs=[urows(half), rows(d), rows(1), full(1, d), full(d, half)],
        out_specs=[rows(d), rows(half)],
        out_shape=[jax.ShapeDtypeStruct((n, d), f32),
                   jax.ShapeDtypeStruct((n, half), f32)],
    )

    def s2_body(u_ref, p2_ref, degp_ref, bd1, p3_ref):
        dis = dis_blk(degp_ref)
        ucat = jnp.concatenate([u_ref[0], u_ref[1]], axis=1)
        h2 = _silu(dis * (ucat + p2_ref[...]) + bd1[...])
        p3_ref[...] = dis * h2

    s2 = pl.pallas_call(
        s2_body,
        grid=grid,
        in_specs=[urows(half // 2), rows(half), rows(1), full(1, half)],
        out_specs=rows(half),
        out_shape=jax.ShapeDtypeStruct((n, half), f32),
    )

    def s3_body(u_ref, p3_ref, h1_ref, degp_ref, wu0, bu0, wu1a, wu1b,
                p4_ref):
        dis = dis_blk(degp_ref)
        ucat = jnp.concatenate([u_ref[0], u_ref[1]], axis=1)
        h3 = _silu(dis * dot(ucat + p3_ref[...], wu0[...]) + bu0[...])
        p4_ref[...] = dis * (dot(h3, wu1a[...]) + dot(h1_ref[...], wu1b[...]))

    s3 = pl.pallas_call(
        s3_body,
        grid=grid,
        in_specs=[urows(half // 2), rows(half), rows(d), rows(1),
                  full(half, d), full(1, d), full(d, d), full(d, d)],
        out_specs=rows(d),
        out_shape=jax.ShapeDtypeStruct((n, d), f32),
    )

    def s4_body(u_ref, p4_ref, degp_ref, bu1, out_ref):
        dis = dis_blk(degp_ref)
        ucat = jnp.concatenate([u_ref[0], u_ref[1]], axis=1)
        out_ref[...] = _silu(dis * (ucat + p4_ref[...]) + bu1[...])

    s4 = pl.pallas_call(
        s4_body,
        grid=grid,
        in_specs=[urows(half), rows(d), rows(1), full(1, d)],
        out_specs=rows(d),
        out_shape=jax.ShapeDtypeStruct((n, d), f32),
    )

    return s0, s1, s2, s3, s4


def kernel(x, noise_graph_X_t, edge_index, t,
           W_t0, b_t0, W_t1, b_t1,
           W_d0, b_d0, W_d1, b_d1,
           W_u0, b_u0, W_u1, b_u1):
    n = x.shape[0]
    d = x.shape[1]
    e = edge_index.shape[1]
    npad = _npad(n)

    chunks = 2 * (-(-e // (NS * ECHUNK * 2)))
    ep = NS * ECHUNK * chunks
    pad = ep - e
    src = jnp.concatenate([edge_index[0], jnp.zeros((pad,), jnp.int32)])
    dst = jnp.concatenate([edge_index[1], jnp.full((pad,), n, jnp.int32)])
    src2 = (2 * src)[None, :] + jnp.arange(2, dtype=jnp.int32)[:, None]
    src2 = src2.reshape(2 * NS, chunks, ECHUNK)
    dst16 = dst.reshape(NS, chunks, ECHUNK)

    chunks32 = -(-e // (NW * ECHUNK))
    epd = NW * ECHUNK * chunks32
    dst32 = jnp.concatenate(
        [edge_index[1], jnp.full((epd - e,), n, jnp.int32)]
    ).reshape(NW, chunks32, ECHUNK)

    zeros64 = jnp.zeros((npad, d // 2), jnp.float32)
    zeros32 = jnp.zeros((npad, d // 4), jnp.float32)

    tpad = -(-2 * n // (NS * 8)) * NS * 8
    agg128 = _make_agg(n, d // 2, chunks)
    agg64 = _make_agg_sp(n, d // 4, chunks, tpad)
    degk = _make_deg(n, chunks32)

    def agg(p, aggk, zeros):
        f = p.shape[1]
        h2 = p.reshape(2 * n, f // 2)
        if aggk is agg64:
            h2 = jnp.pad(h2, ((0, tpad - 2 * n), (0, 0)))
        return aggk(h2, src2, dst16, zeros)

    s0, s1, s2, s3, s4 = _make_dense_stages(n, d)

    degout = degk(dst32)
    degn = (degout[0, :n] + degout[1, :n]).reshape(n, 1)

    row = lambda v: v.reshape(1, -1)
    q1 = s0(t.reshape(n, 1), noise_graph_X_t,
            W_t0, row(b_t0), W_t1, row(b_t1), W_d0)
    p1 = lax.rsqrt(degn + 1.0) * q1
    u1 = agg(p1, agg128, zeros64)
    h1, p2 = s1(u1, p1, degn, row(b_d0), W_d1)
    u2 = agg(p2, agg64, zeros32)
    p3 = s2(u2, p2, degn, row(b_d1))
    u3 = agg(p3, agg64, zeros32)
    p4 = s3(u3, p3, h1, degn, W_u0, row(b_u0), W_u1[:d], W_u1[d:])
    u4 = agg(p4, agg128, zeros64)
    return s4(u4, p4, degn, row(b_u1))
